# Initial kernel scaffold; baseline (speedup 1.0000x reference)
import jax
import jax.numpy as jnp
from jax.experimental import pallas as pl


def kernel(dna_tokens, expr_data, dna_table, pos_enc, expr_W, expr_b, dna_proj_W, dna_proj_b, expr_proj_W, expr_proj_b, ln_gamma, ln_beta):
    raise NotImplementedError("write your pallas kernel here")


# speedup vs baseline: 2.1914x; 2.1914x over previous
import jax
import jax.numpy as jnp
from jax import lax
from jax.experimental import pallas as pl
from jax.experimental.pallas import tpu as pltpu

B, S, V, D, C, E, H = 64, 2048, 6, 128, 40, 64, 512
BLK = 256
NS = S // BLK


def _fused_kernel(tok_ref, pos_ref, tab_ref, w_ref, xd_ref, xw_ref, xb_ref,
                  pw_ref, b2_ref, g_ref, bt_ref, out_ref,
                  tp_ref, eh_ref, pp_ref):
    si = pl.program_id(0)
    bi = pl.program_id(1)

    @pl.when(jnp.logical_and(si == 0, bi == 0))
    def _init():
        tp_ref[...] = lax.dot_general(
            tab_ref[...], w_ref[...], (((1,), (1,)), ((), ())),
            preferred_element_type=jnp.float32)
        e = lax.dot_general(
            xd_ref[...], xw_ref[...], (((1,), (1,)), ((), ())),
            preferred_element_type=jnp.float32) + xb_ref[...]
        eh_ref[...] = lax.dot_general(
            e, pw_ref[...], (((1,), (1,)), ((), ())),
            preferred_element_type=jnp.float32) + b2_ref[...]

    @pl.when(bi == 0)
    def _pos():
        pp_ref[...] = lax.dot_general(
            pos_ref[...], w_ref[...], (((1,), (1,)), ((), ())),
            preferred_element_type=jnp.float32)

    tok = tok_ref[...]
    tp = tp_ref[...]
    x = jnp.where(tok == 1, tp[1:2, :], tp[0:1, :])
    x = jnp.where(tok == 2, tp[2:3, :], x)
    x = jnp.where(tok == 3, tp[3:4, :], x)
    x = jnp.where(tok == 4, tp[4:5, :], x)
    x = jnp.where(tok == 5, tp[5:6, :], x)

    eh_row = eh_ref[pl.ds(bi, 1), :]
    x = x + pp_ref[...] + eh_row

    mu = jnp.mean(x, axis=1, keepdims=True)
    xm = x - mu
    var = jnp.mean(xm * xm, axis=1, keepdims=True)
    r = lax.rsqrt(var + 1e-5)
    out_ref[0] = xm * r * g_ref[...] + bt_ref[...]


def kernel(dna_tokens, expr_data, dna_table, pos_enc, expr_W, expr_b,
           dna_proj_W, dna_proj_b, expr_proj_W, expr_proj_b, ln_gamma, ln_beta):
    tok2 = dna_tokens.astype(jnp.int32).reshape(B * S, 1)
    tab8 = jnp.pad(dna_table, ((0, 8 - V), (0, 0)))
    xd = jnp.pad(expr_data, ((0, 0), (0, 128 - C)))
    xw = jnp.pad(expr_W, ((0, 0), (0, 128 - C)))
    xb = expr_b.reshape(1, E)
    b2 = (expr_proj_b + dna_proj_b).reshape(1, H)
    g2 = ln_gamma.reshape(1, H)
    bt2 = ln_beta.reshape(1, H)

    grid = (NS, B)
    out = pl.pallas_call(
        _fused_kernel,
        grid=grid,
        in_specs=[
            pl.BlockSpec((BLK, 1), lambda si, bi: (bi * NS + si, 0)),
            pl.BlockSpec((BLK, D), lambda si, bi: (si, 0)),
            pl.BlockSpec((8, D), lambda si, bi: (0, 0)),
            pl.BlockSpec((H, D), lambda si, bi: (0, 0)),
            pl.BlockSpec((B, 128), lambda si, bi: (0, 0)),
            pl.BlockSpec((E, 128), lambda si, bi: (0, 0)),
            pl.BlockSpec((1, E), lambda si, bi: (0, 0)),
            pl.BlockSpec((H, E), lambda si, bi: (0, 0)),
            pl.BlockSpec((1, H), lambda si, bi: (0, 0)),
            pl.BlockSpec((1, H), lambda si, bi: (0, 0)),
            pl.BlockSpec((1, H), lambda si, bi: (0, 0)),
        ],
        out_specs=pl.BlockSpec((1, BLK, H), lambda si, bi: (bi, si, 0)),
        out_shape=jax.ShapeDtypeStruct((B, S, H), jnp.float32),
        scratch_shapes=[
            pltpu.VMEM((8, H), jnp.float32),
            pltpu.VMEM((B, H), jnp.float32),
            pltpu.VMEM((BLK, H), jnp.float32),
        ],
    )(tok2, pos_enc, tab8, dna_proj_W, xd, xw, xb, expr_proj_W, b2, g2, bt2)
    return out


# speedup vs baseline: 2.9279x; 1.3361x over previous
import jax
import jax.numpy as jnp
from jax import lax
from jax.experimental import pallas as pl
from jax.experimental.pallas import tpu as pltpu

B, S, V, D, C, E, H = 64, 2048, 6, 128, 40, 64, 512
BLK = 256
NS = S // BLK


def _dotT(a, b):
    return lax.dot_general(a, b, (((1,), (1,)), ((), ())),
                           preferred_element_type=jnp.float32)


def _sel6(masks, operands, init):
    acc = init
    for m, o in zip(masks, operands):
        acc = jnp.where(m, o, acc)
    return acc


def _fused_kernel(tok_lane_ref, tok_all_ref, pos_ref, tab_ref, w_ref,
                  xd_ref, xw_ref, xb_ref, pw_ref, b2_ref, g_ref, bt_ref,
                  out_ref,
                  tp_ref, eh_ref, pp_ref, g2t_ref, meh_ref, qeh_ref,
                  mu_ref, r_ref):
    si = pl.program_id(0)
    bi = pl.program_id(1)

    @pl.when(jnp.logical_and(si == 0, bi == 0))
    def _init():
        tp_ref[...] = _dotT(tab_ref[...], w_ref[...])
        e = _dotT(xd_ref[...], xw_ref[...]) + xb_ref[...]
        eh = _dotT(e, pw_ref[...]) + b2_ref[...]
        eh_ref[...] = eh
        g2t_ref[...] = _dotT(eh, tp_ref[...])
        meh_ref[...] = jnp.mean(eh, axis=1, keepdims=True)
        qeh_ref[...] = jnp.sum(eh * eh, axis=1, keepdims=True)

    @pl.when(bi == 0)
    def _per_sblock():
        tp = tp_ref[...]
        pp = _dotT(pos_ref[...], w_ref[...])
        pp_ref[...] = pp
        g1 = _dotT(tp, pp)
        g3 = _dotT(pp, eh_ref[...])
        tok_all = tok_all_ref[...]
        masks = [tok_all == v for v in range(1, V)]
        mtp = jnp.mean(tp, axis=1, keepdims=True)
        qtp = jnp.sum(tp * tp, axis=1, keepdims=True)
        g2t = g2t_ref[...]
        mu_pre = _sel6(masks, [mtp[v:v + 1, 0:1] for v in range(1, V)],
                       mtp[0:1, 0:1]) + meh_ref[...]
        ss_pre = (_sel6(masks, [qtp[v:v + 1, 0:1] for v in range(1, V)],
                        qtp[0:1, 0:1])
                  + 2.0 * _sel6(masks, [g1[v:v + 1, :] for v in range(1, V)],
                                g1[0:1, :])
                  + 2.0 * _sel6(masks, [g2t[:, v:v + 1] for v in range(1, V)],
                                g2t[:, 0:1])
                  + qeh_ref[...])
        mu_t = mu_pre.T
        ss_t = ss_pre.T
        m_pp = jnp.mean(pp, axis=1, keepdims=True)
        q_pp = jnp.sum(pp * pp, axis=1, keepdims=True)
        mu = mu_t + m_pp
        ss = ss_t + q_pp + 2.0 * g3
        var = ss * (1.0 / H) - mu * mu
        mu_ref[...] = mu
        r_ref[...] = lax.rsqrt(var + 1e-5)

    tok = tok_lane_ref[0]
    iota = lax.broadcasted_iota(jnp.int32, (8, BLK), 0)
    oh = (iota == tok).astype(jnp.float32)
    x = lax.dot_general(oh, tp_ref[...], (((0,), (0,)), ((), ())),
                        preferred_element_type=jnp.float32)
    x = x + pp_ref[...] + eh_ref[pl.ds(bi, 1), :]
    ohb = (lax.broadcasted_iota(jnp.int32, (1, B), 1) == bi).astype(jnp.float32)
    mu_col = jnp.sum(mu_ref[...] * ohb, axis=1, keepdims=True)
    r_col = jnp.sum(r_ref[...] * ohb, axis=1, keepdims=True)
    out_ref[0] = ((x - mu_col) * r_col) * g_ref[...] + bt_ref[...]


def kernel(dna_tokens, expr_data, dna_table, pos_enc, expr_W, expr_b,
           dna_proj_W, dna_proj_b, expr_proj_W, expr_proj_b, ln_gamma, ln_beta):
    toks = dna_tokens.astype(jnp.int32)
    tok_lane = toks.reshape(B * NS, 1, BLK)
    tab8 = jnp.pad(dna_table, ((0, 8 - V), (0, 0)))
    xd = jnp.pad(expr_data, ((0, 0), (0, 128 - C)))
    xw = jnp.pad(expr_W, ((0, 0), (0, 128 - C)))
    xb = expr_b.reshape(1, E)
    b2 = (expr_proj_b + dna_proj_b).reshape(1, H)
    g2 = ln_gamma.reshape(1, H)
    bt2 = ln_beta.reshape(1, H)

    grid = (NS, B)
    out = pl.pallas_call(
        _fused_kernel,
        grid=grid,
        in_specs=[
            pl.BlockSpec((1, 1, BLK), lambda si, bi: (bi * NS + si, 0, 0)),
            pl.BlockSpec((B, BLK), lambda si, bi: (0, si)),
            pl.BlockSpec((BLK, D), lambda si, bi: (si, 0)),
            pl.BlockSpec((8, D), lambda si, bi: (0, 0)),
            pl.BlockSpec((H, D), lambda si, bi: (0, 0)),
            pl.BlockSpec((B, 128), lambda si, bi: (0, 0)),
            pl.BlockSpec((E, 128), lambda si, bi: (0, 0)),
            pl.BlockSpec((1, E), lambda si, bi: (0, 0)),
            pl.BlockSpec((H, E), lambda si, bi: (0, 0)),
            pl.BlockSpec((1, H), lambda si, bi: (0, 0)),
            pl.BlockSpec((1, H), lambda si, bi: (0, 0)),
            pl.BlockSpec((1, H), lambda si, bi: (0, 0)),
        ],
        out_specs=pl.BlockSpec((1, BLK, H), lambda si, bi: (bi, si, 0)),
        out_shape=jax.ShapeDtypeStruct((B, S, H), jnp.float32),
        scratch_shapes=[
            pltpu.VMEM((8, H), jnp.float32),
            pltpu.VMEM((B, H), jnp.float32),
            pltpu.VMEM((BLK, H), jnp.float32),
            pltpu.VMEM((B, 8), jnp.float32),
            pltpu.VMEM((B, 1), jnp.float32),
            pltpu.VMEM((B, 1), jnp.float32),
            pltpu.VMEM((BLK, B), jnp.float32),
            pltpu.VMEM((BLK, B), jnp.float32),
        ],
    )(tok_lane, toks, pos_enc, tab8, dna_proj_W, xd, xw, xb, expr_proj_W,
      b2, g2, bt2)
    return out


# speedup vs baseline: 2.9990x; 1.0243x over previous
import jax
import jax.numpy as jnp
from jax import lax
from jax.experimental import pallas as pl
from jax.experimental.pallas import tpu as pltpu

B, S, V, D, C, E, H = 64, 2048, 6, 128, 40, 64, 512
BLK = 256
NS = S // BLK


def _dotT(a, b):
    return lax.dot_general(a, b, (((1,), (1,)), ((), ())),
                           preferred_element_type=jnp.float32)


def _sel6(masks, operands, init):
    acc = init
    for m, o in zip(masks, operands):
        acc = jnp.where(m, o, acc)
    return acc


def _fused_kernel(tok_lane_ref, tok_all_ref, pos_ref, tab_ref, w_ref,
                  xd_ref, xw_ref, xb_ref, pw_ref, b2_ref, g_ref, bt_ref,
                  out_ref,
                  tp_ref, eh_ref, ehg_ref, ppg_ref, g2t_ref, meh_ref,
                  qeh_ref, mr_ref, r_ref, rt_ref, lhs_ref, rhs_ref):
    si = pl.program_id(0)
    bi = pl.program_id(1)

    @pl.when(jnp.logical_and(si == 0, bi == 0))
    def _init():
        g = g_ref[...]
        tp = _dotT(tab_ref[...], w_ref[...])
        tp_ref[...] = tp
        e = _dotT(xd_ref[...], xw_ref[...]) + xb_ref[...]
        eh = _dotT(e, pw_ref[...]) + b2_ref[...]
        eh_ref[...] = eh
        ehg_ref[...] = eh * g
        g2t_ref[...] = _dotT(eh, tp)
        meh_ref[...] = jnp.mean(eh, axis=1, keepdims=True)
        qeh_ref[...] = jnp.sum(eh * eh, axis=1, keepdims=True)
        rhs_ref[0:8, :] = tp * g
        rhs_ref[8:9, :] = -g
        rhs_ref[9:10, :] = bt_ref[...]
        rhs_ref[10:16, :] = jnp.zeros((6, H), jnp.float32)
        lhs_ref[8:16, :] = jnp.zeros((8, BLK), jnp.float32)
        lhs_ref[9:10, :] = jnp.ones((1, BLK), jnp.float32)

    @pl.when(bi == 0)
    def _per_sblock():
        tp = tp_ref[...]
        eh = eh_ref[...]
        pp = _dotT(pos_ref[...], w_ref[...])
        ppg_ref[...] = pp * g_ref[...]
        g1 = _dotT(tp, pp)
        g3t = _dotT(eh, pp)
        ones_h = jnp.ones((1, H), jnp.float32)
        m_pp = _dotT(ones_h, pp) * (1.0 / H)
        q_pp = _dotT(ones_h, pp * pp)
        tok_all = tok_all_ref[...]
        masks = [tok_all == v for v in range(1, V)]
        mtp = jnp.mean(tp, axis=1, keepdims=True)
        qtp = jnp.sum(tp * tp, axis=1, keepdims=True)
        g2t = g2t_ref[...]
        mu = (_sel6(masks, [mtp[v:v + 1, 0:1] for v in range(1, V)],
                    mtp[0:1, 0:1])
              + meh_ref[...] + m_pp)
        ss = (_sel6(masks, [qtp[v:v + 1, 0:1] for v in range(1, V)],
                    qtp[0:1, 0:1])
              + 2.0 * _sel6(masks, [g1[v:v + 1, :] for v in range(1, V)],
                            g1[0:1, :])
              + 2.0 * _sel6(masks, [g2t[:, v:v + 1] for v in range(1, V)],
                            g2t[:, 0:1])
              + qeh_ref[...] + q_pp + 2.0 * g3t)
        var = ss * (1.0 / H) - mu * mu
        r = lax.rsqrt(var + 1e-5)
        r_ref[...] = r
        mr_ref[...] = mu * r
        rt_ref[...] = r.T

    tok = tok_lane_ref[0]
    iota = lax.broadcasted_iota(jnp.int32, (8, BLK), 0)
    r_row = r_ref[pl.ds(bi, 1), :]
    lhs_ref[0:8, :] = jnp.where(iota == tok, r_row, 0.0)
    lhs_ref[8:9, :] = mr_ref[pl.ds(bi, 1), :]
    y = lax.dot_general(lhs_ref[...], rhs_ref[...],
                        (((0,), (0,)), ((), ())),
                        preferred_element_type=jnp.float32)
    ohb = (lax.broadcasted_iota(jnp.int32, (1, B), 1) == bi).astype(jnp.float32)
    r_col = jnp.sum(rt_ref[...] * ohb, axis=1, keepdims=True)
    t = ppg_ref[...] + ehg_ref[pl.ds(bi, 1), :]
    out_ref[0] = y + t * r_col


def kernel(dna_tokens, expr_data, dna_table, pos_enc, expr_W, expr_b,
           dna_proj_W, dna_proj_b, expr_proj_W, expr_proj_b, ln_gamma, ln_beta):
    toks = dna_tokens.astype(jnp.int32)
    tok_lane = toks.reshape(B * NS, 1, BLK)
    tab8 = jnp.pad(dna_table, ((0, 8 - V), (0, 0)))
    xd = jnp.pad(expr_data, ((0, 0), (0, 128 - C)))
    xw = jnp.pad(expr_W, ((0, 0), (0, 128 - C)))
    xb = expr_b.reshape(1, E)
    b2 = (expr_proj_b + dna_proj_b).reshape(1, H)
    g2 = ln_gamma.reshape(1, H)
    bt2 = ln_beta.reshape(1, H)

    grid = (NS, B)
    out = pl.pallas_call(
        _fused_kernel,
        grid=grid,
        in_specs=[
            pl.BlockSpec((1, 1, BLK), lambda si, bi: (bi * NS + si, 0, 0)),
            pl.BlockSpec((B, BLK), lambda si, bi: (0, si)),
            pl.BlockSpec((BLK, D), lambda si, bi: (si, 0)),
            pl.BlockSpec((8, D), lambda si, bi: (0, 0)),
            pl.BlockSpec((H, D), lambda si, bi: (0, 0)),
            pl.BlockSpec((B, 128), lambda si, bi: (0, 0)),
            pl.BlockSpec((E, 128), lambda si, bi: (0, 0)),
            pl.BlockSpec((1, E), lambda si, bi: (0, 0)),
            pl.BlockSpec((H, E), lambda si, bi: (0, 0)),
            pl.BlockSpec((1, H), lambda si, bi: (0, 0)),
            pl.BlockSpec((1, H), lambda si, bi: (0, 0)),
            pl.BlockSpec((1, H), lambda si, bi: (0, 0)),
        ],
        out_specs=pl.BlockSpec((1, BLK, H), lambda si, bi: (bi, si, 0)),
        out_shape=jax.ShapeDtypeStruct((B, S, H), jnp.float32),
        scratch_shapes=[
            pltpu.VMEM((8, H), jnp.float32),
            pltpu.VMEM((B, H), jnp.float32),
            pltpu.VMEM((B, H), jnp.float32),
            pltpu.VMEM((BLK, H), jnp.float32),
            pltpu.VMEM((B, 8), jnp.float32),
            pltpu.VMEM((B, 1), jnp.float32),
            pltpu.VMEM((B, 1), jnp.float32),
            pltpu.VMEM((B, BLK), jnp.float32),
            pltpu.VMEM((B, BLK), jnp.float32),
            pltpu.VMEM((BLK, B), jnp.float32),
            pltpu.VMEM((16, BLK), jnp.float32),
            pltpu.VMEM((16, H), jnp.float32),
        ],
    )(tok_lane, toks, pos_enc, tab8, dna_proj_W, xd, xw, xb, expr_proj_W,
      b2, g2, bt2)
    return out


# speedup vs baseline: 4.7838x; 1.5951x over previous
import jax
import jax.numpy as jnp
from jax import lax
from jax.experimental import pallas as pl
from jax.experimental.pallas import tpu as pltpu

B, S, V, D, C, E, H = 64, 2048, 6, 128, 40, 64, 512
BLK = 512
NS = S // BLK


def _dotT(a, b):
    return lax.dot_general(a, b, (((1,), (1,)), ((), ())),
                           preferred_element_type=jnp.float32)


def _sel6(masks, operands, init):
    acc = init
    for m, o in zip(masks, operands):
        acc = jnp.where(m, o, acc)
    return acc


def _fused_kernel(tok_lane_ref, tok_all_ref, pos_ref, tab_ref, w_ref,
                  xd_ref, xw_ref, xb_ref, pw_ref, b2_ref, g_ref, bt_ref,
                  out_ref,
                  tp_ref, eh_ref, ehg_ref, ppg_ref, g2t_ref, meh_ref,
                  qeh_ref, mr_ref, r_ref, rt_ref, lhs_ref, rhs_ref):
    si = pl.program_id(0)
    bi = pl.program_id(1)

    @pl.when(jnp.logical_and(si == 0, bi == 0))
    def _init():
        g = g_ref[...]
        tp = _dotT(tab_ref[...], w_ref[...])
        tp_ref[...] = tp
        e = _dotT(xd_ref[...], xw_ref[...]) + xb_ref[...]
        eh = _dotT(e, pw_ref[...]) + b2_ref[...]
        eh_ref[...] = eh
        ehg_ref[...] = eh * g
        g2t_ref[...] = _dotT(eh, tp)
        meh_ref[...] = jnp.mean(eh, axis=1, keepdims=True)
        qeh_ref[...] = jnp.sum(eh * eh, axis=1, keepdims=True)
        rhs_ref[0:8, :] = tp * g
        rhs_ref[8:9, :] = -g
        rhs_ref[9:10, :] = bt_ref[...]
        rhs_ref[10:16, :] = jnp.zeros((6, H), jnp.float32)
        lhs_ref[8:16, :] = jnp.zeros((8, BLK), jnp.float32)
        lhs_ref[9:10, :] = jnp.ones((1, BLK), jnp.float32)

    @pl.when(bi == 0)
    def _per_sblock():
        tp = tp_ref[...]
        eh = eh_ref[...]
        pp = _dotT(pos_ref[...], w_ref[...])
        ppg_ref[...] = pp * g_ref[...]
        g1 = _dotT(tp, pp)
        g3t = _dotT(eh, pp)
        ones_h = jnp.ones((1, H), jnp.float32)
        m_pp = _dotT(ones_h, pp) * (1.0 / H)
        q_pp = _dotT(ones_h, pp * pp)
        tok_all = tok_all_ref[...]
        masks = [tok_all == v for v in range(1, V)]
        mtp = jnp.mean(tp, axis=1, keepdims=True)
        qtp = jnp.sum(tp * tp, axis=1, keepdims=True)
        g2t = g2t_ref[...]
        mu = (_sel6(masks, [mtp[v:v + 1, 0:1] for v in range(1, V)],
                    mtp[0:1, 0:1])
              + meh_ref[...] + m_pp)
        ss = (_sel6(masks, [qtp[v:v + 1, 0:1] for v in range(1, V)],
                    qtp[0:1, 0:1])
              + 2.0 * _sel6(masks, [g1[v:v + 1, :] for v in range(1, V)],
                            g1[0:1, :])
              + 2.0 * _sel6(masks, [g2t[:, v:v + 1] for v in range(1, V)],
                            g2t[:, 0:1])
              + qeh_ref[...] + q_pp + 2.0 * g3t)
        var = ss * (1.0 / H) - mu * mu
        r = lax.rsqrt(var + 1e-5)
        r_ref[...] = r
        mr_ref[...] = mu * r
        rt_ref[...] = r.T

    tok = tok_lane_ref[0]
    iota = lax.broadcasted_iota(jnp.int32, (8, BLK), 0)
    r_row = r_ref[pl.ds(bi, 1), :]
    lhs_ref[0:8, :] = jnp.where(iota == tok, r_row, 0.0)
    lhs_ref[8:9, :] = mr_ref[pl.ds(bi, 1), :]
    y = lax.dot_general(lhs_ref[...], rhs_ref[...],
                        (((0,), (0,)), ((), ())),
                        preferred_element_type=jnp.float32)
    ohb = (lax.broadcasted_iota(jnp.int32, (1, B), 1) == bi).astype(jnp.float32)
    r_col = jnp.sum(rt_ref[...] * ohb, axis=1, keepdims=True)
    t = ppg_ref[...] + ehg_ref[pl.ds(bi, 1), :]
    out_ref[0] = y + t * r_col


def kernel(dna_tokens, expr_data, dna_table, pos_enc, expr_W, expr_b,
           dna_proj_W, dna_proj_b, expr_proj_W, expr_proj_b, ln_gamma, ln_beta):
    toks = dna_tokens.astype(jnp.int32)
    tok_lane = toks.reshape(B * NS, 1, BLK)
    tab8 = jnp.pad(dna_table, ((0, 8 - V), (0, 0)))
    xd = jnp.pad(expr_data, ((0, 0), (0, 128 - C)))
    xw = jnp.pad(expr_W, ((0, 0), (0, 128 - C)))
    xb = expr_b.reshape(1, E)
    b2 = (expr_proj_b + dna_proj_b).reshape(1, H)
    g2 = ln_gamma.reshape(1, H)
    bt2 = ln_beta.reshape(1, H)

    grid = (NS, B)
    out = pl.pallas_call(
        _fused_kernel,
        grid=grid,
        in_specs=[
            pl.BlockSpec((1, 1, BLK), lambda si, bi: (bi * NS + si, 0, 0)),
            pl.BlockSpec((B, BLK), lambda si, bi: (0, si)),
            pl.BlockSpec((BLK, D), lambda si, bi: (si, 0)),
            pl.BlockSpec((8, D), lambda si, bi: (0, 0)),
            pl.BlockSpec((H, D), lambda si, bi: (0, 0)),
            pl.BlockSpec((B, 128), lambda si, bi: (0, 0)),
            pl.BlockSpec((E, 128), lambda si, bi: (0, 0)),
            pl.BlockSpec((1, E), lambda si, bi: (0, 0)),
            pl.BlockSpec((H, E), lambda si, bi: (0, 0)),
            pl.BlockSpec((1, H), lambda si, bi: (0, 0)),
            pl.BlockSpec((1, H), lambda si, bi: (0, 0)),
            pl.BlockSpec((1, H), lambda si, bi: (0, 0)),
        ],
        out_specs=pl.BlockSpec((1, BLK, H), lambda si, bi: (bi, si, 0)),
        out_shape=jax.ShapeDtypeStruct((B, S, H), jnp.float32),
        scratch_shapes=[
            pltpu.VMEM((8, H), jnp.float32),
            pltpu.VMEM((B, H), jnp.float32),
            pltpu.VMEM((B, H), jnp.float32),
            pltpu.VMEM((BLK, H), jnp.float32),
            pltpu.VMEM((B, 8), jnp.float32),
            pltpu.VMEM((B, 1), jnp.float32),
            pltpu.VMEM((B, 1), jnp.float32),
            pltpu.VMEM((B, BLK), jnp.float32),
            pltpu.VMEM((B, BLK), jnp.float32),
            pltpu.VMEM((BLK, B), jnp.float32),
            pltpu.VMEM((16, BLK), jnp.float32),
            pltpu.VMEM((16, H), jnp.float32),
        ],
    )(tok_lane, toks, pos_enc, tab8, dna_proj_W, xd, xw, xb, expr_proj_W,
      b2, g2, bt2)
    return out


# speedup vs baseline: 6.9039x; 1.4432x over previous
import jax
import jax.numpy as jnp
from jax import lax
from jax.experimental import pallas as pl
from jax.experimental.pallas import tpu as pltpu

B, S, V, D, C, E, H = 64, 2048, 6, 128, 40, 64, 512
BLK = 1024
NS = S // BLK


def _dotT(a, b):
    return lax.dot_general(a, b, (((1,), (1,)), ((), ())),
                           preferred_element_type=jnp.float32)


def _sel6(masks, operands, init):
    acc = init
    for m, o in zip(masks, operands):
        acc = jnp.where(m, o, acc)
    return acc


def _fused_kernel(tok_lane_ref, tok_all_ref, pos_ref, tab_ref, w_ref,
                  xd_ref, xw_ref, xb_ref, pw_ref, b2_ref, g_ref, bt_ref,
                  out_ref,
                  tp_ref, eh_ref, ehg_ref, ppg_ref, g2t_ref, meh_ref,
                  qeh_ref, mr_ref, r_ref, rt_ref, lhs_ref, rhs_ref):
    si = pl.program_id(0)
    bi = pl.program_id(1)

    @pl.when(jnp.logical_and(si == 0, bi == 0))
    def _init():
        g = g_ref[...]
        tp = _dotT(tab_ref[...], w_ref[...])
        tp_ref[...] = tp
        e = _dotT(xd_ref[...], xw_ref[...]) + xb_ref[...]
        eh = _dotT(e, pw_ref[...]) + b2_ref[...]
        eh_ref[...] = eh
        ehg_ref[...] = eh * g
        g2t_ref[...] = _dotT(eh, tp)
        meh_ref[...] = jnp.mean(eh, axis=1, keepdims=True)
        qeh_ref[...] = jnp.sum(eh * eh, axis=1, keepdims=True)
        rhs_ref[0:8, :] = tp * g
        rhs_ref[8:9, :] = -g
        rhs_ref[9:10, :] = bt_ref[...]
        rhs_ref[10:16, :] = jnp.zeros((6, H), jnp.float32)
        lhs_ref[8:16, :] = jnp.zeros((8, BLK), jnp.float32)
        lhs_ref[9:10, :] = jnp.ones((1, BLK), jnp.float32)

    @pl.when(bi == 0)
    def _per_sblock():
        tp = tp_ref[...]
        eh = eh_ref[...]
        pp = _dotT(pos_ref[...], w_ref[...])
        ppg_ref[...] = pp * g_ref[...]
        g1 = _dotT(tp, pp)
        g3t = _dotT(eh, pp)
        ones_h = jnp.ones((1, H), jnp.float32)
        m_pp = _dotT(ones_h, pp) * (1.0 / H)
        q_pp = _dotT(ones_h, pp * pp)
        tok_all = tok_all_ref[...]
        masks = [tok_all == v for v in range(1, V)]
        mtp = jnp.mean(tp, axis=1, keepdims=True)
        qtp = jnp.sum(tp * tp, axis=1, keepdims=True)
        g2t = g2t_ref[...]
        mu = (_sel6(masks, [mtp[v:v + 1, 0:1] for v in range(1, V)],
                    mtp[0:1, 0:1])
              + meh_ref[...] + m_pp)
        ss = (_sel6(masks, [qtp[v:v + 1, 0:1] for v in range(1, V)],
                    qtp[0:1, 0:1])
              + 2.0 * _sel6(masks, [g1[v:v + 1, :] for v in range(1, V)],
                            g1[0:1, :])
              + 2.0 * _sel6(masks, [g2t[:, v:v + 1] for v in range(1, V)],
                            g2t[:, 0:1])
              + qeh_ref[...] + q_pp + 2.0 * g3t)
        var = ss * (1.0 / H) - mu * mu
        r = lax.rsqrt(var + 1e-5)
        r_ref[...] = r
        mr_ref[...] = mu * r
        rt_ref[...] = r.T

    tok = tok_lane_ref[0]
    iota = lax.broadcasted_iota(jnp.int32, (8, BLK), 0)
    r_row = r_ref[pl.ds(bi, 1), :]
    lhs_ref[0:8, :] = jnp.where(iota == tok, r_row, 0.0)
    lhs_ref[8:9, :] = mr_ref[pl.ds(bi, 1), :]
    y = lax.dot_general(lhs_ref[...], rhs_ref[...],
                        (((0,), (0,)), ((), ())),
                        preferred_element_type=jnp.float32)
    ohb = (lax.broadcasted_iota(jnp.int32, (1, B), 1) == bi).astype(jnp.float32)
    r_col = jnp.sum(rt_ref[...] * ohb, axis=1, keepdims=True)
    t = ppg_ref[...] + ehg_ref[pl.ds(bi, 1), :]
    out_ref[0] = y + t * r_col


def kernel(dna_tokens, expr_data, dna_table, pos_enc, expr_W, expr_b,
           dna_proj_W, dna_proj_b, expr_proj_W, expr_proj_b, ln_gamma, ln_beta):
    toks = dna_tokens.astype(jnp.int32)
    tok_lane = toks.reshape(B * NS, 1, BLK)
    tab8 = jnp.pad(dna_table, ((0, 8 - V), (0, 0)))
    xd = jnp.pad(expr_data, ((0, 0), (0, 128 - C)))
    xw = jnp.pad(expr_W, ((0, 0), (0, 128 - C)))
    xb = expr_b.reshape(1, E)
    b2 = (expr_proj_b + dna_proj_b).reshape(1, H)
    g2 = ln_gamma.reshape(1, H)
    bt2 = ln_beta.reshape(1, H)

    grid = (NS, B)
    out = pl.pallas_call(
        _fused_kernel,
        grid=grid,
        in_specs=[
            pl.BlockSpec((1, 1, BLK), lambda si, bi: (bi * NS + si, 0, 0)),
            pl.BlockSpec((B, BLK), lambda si, bi: (0, si)),
            pl.BlockSpec((BLK, D), lambda si, bi: (si, 0)),
            pl.BlockSpec((8, D), lambda si, bi: (0, 0)),
            pl.BlockSpec((H, D), lambda si, bi: (0, 0)),
            pl.BlockSpec((B, 128), lambda si, bi: (0, 0)),
            pl.BlockSpec((E, 128), lambda si, bi: (0, 0)),
            pl.BlockSpec((1, E), lambda si, bi: (0, 0)),
            pl.BlockSpec((H, E), lambda si, bi: (0, 0)),
            pl.BlockSpec((1, H), lambda si, bi: (0, 0)),
            pl.BlockSpec((1, H), lambda si, bi: (0, 0)),
            pl.BlockSpec((1, H), lambda si, bi: (0, 0)),
        ],
        out_specs=pl.BlockSpec((1, BLK, H), lambda si, bi: (bi, si, 0)),
        out_shape=jax.ShapeDtypeStruct((B, S, H), jnp.float32),
        scratch_shapes=[
            pltpu.VMEM((8, H), jnp.float32),
            pltpu.VMEM((B, H), jnp.float32),
            pltpu.VMEM((B, H), jnp.float32),
            pltpu.VMEM((BLK, H), jnp.float32),
            pltpu.VMEM((B, 8), jnp.float32),
            pltpu.VMEM((B, 1), jnp.float32),
            pltpu.VMEM((B, 1), jnp.float32),
            pltpu.VMEM((B, BLK), jnp.float32),
            pltpu.VMEM((B, BLK), jnp.float32),
            pltpu.VMEM((BLK, B), jnp.float32),
            pltpu.VMEM((16, BLK), jnp.float32),
            pltpu.VMEM((16, H), jnp.float32),
        ],
    )(tok_lane, toks, pos_enc, tab8, dna_proj_W, xd, xw, xb, expr_proj_W,
      b2, g2, bt2)
    return out


# speedup vs baseline: 8.8163x; 1.2770x over previous
import jax
import jax.numpy as jnp
from jax import lax
from jax.experimental import pallas as pl
from jax.experimental.pallas import tpu as pltpu

B, S, V, D, C, E, H = 64, 2048, 6, 128, 40, 64, 512
BLK = 2048
NS = S // BLK


def _dotT(a, b):
    return lax.dot_general(a, b, (((1,), (1,)), ((), ())),
                           preferred_element_type=jnp.float32)


def _sel6(masks, operands, init):
    acc = init
    for m, o in zip(masks, operands):
        acc = jnp.where(m, o, acc)
    return acc


def _fused_kernel(tok_lane_ref, tok_all_ref, pos_ref, tab_ref, w_ref,
                  xd_ref, xw_ref, xb_ref, pw_ref, b2_ref, g_ref, bt_ref,
                  out_ref,
                  tp_ref, eh_ref, ehg_ref, ppg_ref, g2t_ref, meh_ref,
                  qeh_ref, mr_ref, r_ref, rt_ref, lhs_ref, rhs_ref):
    si = pl.program_id(0)
    bi = pl.program_id(1)

    @pl.when(jnp.logical_and(si == 0, bi == 0))
    def _init():
        g = g_ref[...]
        tp = _dotT(tab_ref[...], w_ref[...])
        tp_ref[...] = tp
        e = _dotT(xd_ref[...], xw_ref[...]) + xb_ref[...]
        eh = _dotT(e, pw_ref[...]) + b2_ref[...]
        eh_ref[...] = eh
        ehg_ref[...] = eh * g
        g2t_ref[...] = _dotT(eh, tp)
        meh_ref[...] = jnp.mean(eh, axis=1, keepdims=True)
        qeh_ref[...] = jnp.sum(eh * eh, axis=1, keepdims=True)
        rhs_ref[0:8, :] = tp * g
        rhs_ref[8:9, :] = -g
        rhs_ref[9:10, :] = bt_ref[...]
        rhs_ref[10:16, :] = jnp.zeros((6, H), jnp.float32)
        lhs_ref[8:16, :] = jnp.zeros((8, BLK), jnp.float32)
        lhs_ref[9:10, :] = jnp.ones((1, BLK), jnp.float32)

    @pl.when(bi == 0)
    def _per_sblock():
        tp = tp_ref[...]
        eh = eh_ref[...]
        pp = _dotT(pos_ref[...], w_ref[...])
        ppg_ref[...] = pp * g_ref[...]
        g1 = _dotT(tp, pp)
        g3t = _dotT(eh, pp)
        ones_h = jnp.ones((1, H), jnp.float32)
        m_pp = _dotT(ones_h, pp) * (1.0 / H)
        q_pp = _dotT(ones_h, pp * pp)
        tok_all = tok_all_ref[...]
        masks = [tok_all == v for v in range(1, V)]
        mtp = jnp.mean(tp, axis=1, keepdims=True)
        qtp = jnp.sum(tp * tp, axis=1, keepdims=True)
        g2t = g2t_ref[...]
        mu = (_sel6(masks, [mtp[v:v + 1, 0:1] for v in range(1, V)],
                    mtp[0:1, 0:1])
              + meh_ref[...] + m_pp)
        ss = (_sel6(masks, [qtp[v:v + 1, 0:1] for v in range(1, V)],
                    qtp[0:1, 0:1])
              + 2.0 * _sel6(masks, [g1[v:v + 1, :] for v in range(1, V)],
                            g1[0:1, :])
              + 2.0 * _sel6(masks, [g2t[:, v:v + 1] for v in range(1, V)],
                            g2t[:, 0:1])
              + qeh_ref[...] + q_pp + 2.0 * g3t)
        var = ss * (1.0 / H) - mu * mu
        r = lax.rsqrt(var + 1e-5)
        r_ref[...] = r
        mr_ref[...] = mu * r
        rt_ref[...] = r.T

    tok = tok_lane_ref[0]
    iota = lax.broadcasted_iota(jnp.int32, (8, BLK), 0)
    r_row = r_ref[pl.ds(bi, 1), :]
    lhs_ref[0:8, :] = jnp.where(iota == tok, r_row, 0.0)
    lhs_ref[8:9, :] = mr_ref[pl.ds(bi, 1), :]
    y = lax.dot_general(lhs_ref[...], rhs_ref[...],
                        (((0,), (0,)), ((), ())),
                        preferred_element_type=jnp.float32)
    ohb = (lax.broadcasted_iota(jnp.int32, (1, B), 1) == bi).astype(jnp.float32)
    r_col = jnp.sum(rt_ref[...] * ohb, axis=1, keepdims=True)
    t = ppg_ref[...] + ehg_ref[pl.ds(bi, 1), :]
    out_ref[0] = y + t * r_col


def kernel(dna_tokens, expr_data, dna_table, pos_enc, expr_W, expr_b,
           dna_proj_W, dna_proj_b, expr_proj_W, expr_proj_b, ln_gamma, ln_beta):
    toks = dna_tokens.astype(jnp.int32)
    tok_lane = toks.reshape(B * NS, 1, BLK)
    tab8 = jnp.pad(dna_table, ((0, 8 - V), (0, 0)))
    xd = jnp.pad(expr_data, ((0, 0), (0, 128 - C)))
    xw = jnp.pad(expr_W, ((0, 0), (0, 128 - C)))
    xb = expr_b.reshape(1, E)
    b2 = (expr_proj_b + dna_proj_b).reshape(1, H)
    g2 = ln_gamma.reshape(1, H)
    bt2 = ln_beta.reshape(1, H)

    grid = (NS, B)
    out = pl.pallas_call(
        _fused_kernel,
        grid=grid,
        in_specs=[
            pl.BlockSpec((1, 1, BLK), lambda si, bi: (bi * NS + si, 0, 0)),
            pl.BlockSpec((B, BLK), lambda si, bi: (0, si)),
            pl.BlockSpec((BLK, D), lambda si, bi: (si, 0)),
            pl.BlockSpec((8, D), lambda si, bi: (0, 0)),
            pl.BlockSpec((H, D), lambda si, bi: (0, 0)),
            pl.BlockSpec((B, 128), lambda si, bi: (0, 0)),
            pl.BlockSpec((E, 128), lambda si, bi: (0, 0)),
            pl.BlockSpec((1, E), lambda si, bi: (0, 0)),
            pl.BlockSpec((H, E), lambda si, bi: (0, 0)),
            pl.BlockSpec((1, H), lambda si, bi: (0, 0)),
            pl.BlockSpec((1, H), lambda si, bi: (0, 0)),
            pl.BlockSpec((1, H), lambda si, bi: (0, 0)),
        ],
        out_specs=pl.BlockSpec((1, BLK, H), lambda si, bi: (bi, si, 0)),
        out_shape=jax.ShapeDtypeStruct((B, S, H), jnp.float32),
        scratch_shapes=[
            pltpu.VMEM((8, H), jnp.float32),
            pltpu.VMEM((B, H), jnp.float32),
            pltpu.VMEM((B, H), jnp.float32),
            pltpu.VMEM((BLK, H), jnp.float32),
            pltpu.VMEM((B, 8), jnp.float32),
            pltpu.VMEM((B, 1), jnp.float32),
            pltpu.VMEM((B, 1), jnp.float32),
            pltpu.VMEM((B, BLK), jnp.float32),
            pltpu.VMEM((B, BLK), jnp.float32),
            pltpu.VMEM((BLK, B), jnp.float32),
            pltpu.VMEM((16, BLK), jnp.float32),
            pltpu.VMEM((16, H), jnp.float32),
        ],
    )(tok_lane, toks, pos_enc, tab8, dna_proj_W, xd, xw, xb, expr_proj_W,
      b2, g2, bt2)
    return out


# speedup vs baseline: 9.0584x; 1.0275x over previous
import jax
import jax.numpy as jnp
from jax import lax
from jax.experimental import pallas as pl
from jax.experimental.pallas import tpu as pltpu

B, S, V, D, C, E, H = 64, 2048, 6, 128, 40, 64, 512
BLK = 2048
NS = S // BLK


def _dotT(a, b):
    return lax.dot_general(a, b, (((1,), (1,)), ((), ())),
                           preferred_element_type=jnp.float32)


def _sel6(masks, operands, init):
    acc = init
    for m, o in zip(masks, operands):
        acc = jnp.where(m, o, acc)
    return acc


def _fused_kernel(tok_lane_ref, tok_all_ref, pos_ref, tab_ref, w_ref,
                  xd_ref, xw_ref, xb_ref, pw_ref, b2_ref, g_ref, bt_ref,
                  out_ref,
                  tp_ref, eh_ref, ehg_ref, ppg_ref, g2t_ref, meh_ref,
                  qeh_ref, mr_ref, r_ref, rt_ref, lhs_ref, rhs_ref):
    si = pl.program_id(0)
    bi = pl.program_id(1)

    @pl.when(jnp.logical_and(si == 0, bi == 0))
    def _init():
        g = g_ref[...]
        tp = _dotT(tab_ref[...], w_ref[...])
        tp_ref[...] = tp
        e = _dotT(xd_ref[...], xw_ref[...]) + xb_ref[...]
        eh = _dotT(e, pw_ref[...]) + b2_ref[...]
        eh_ref[...] = eh
        ehg_ref[...] = eh * g
        g2t_ref[...] = _dotT(eh, tp)
        meh_ref[...] = jnp.mean(eh, axis=1, keepdims=True)
        qeh_ref[...] = jnp.sum(eh * eh, axis=1, keepdims=True)
        rhs_ref[0:8, :] = tp * g
        rhs_ref[8:9, :] = -g
        rhs_ref[9:10, :] = bt_ref[...]
        rhs_ref[10:16, :] = jnp.zeros((6, H), jnp.float32)
        lhs_ref[8:16, :] = jnp.zeros((8, BLK), jnp.float32)
        lhs_ref[9:10, :] = jnp.ones((1, BLK), jnp.float32)

    @pl.when(bi == 0)
    def _per_sblock():
        tp = tp_ref[...]
        eh = eh_ref[...]
        pp = _dotT(pos_ref[...], w_ref[...])
        ppg_ref[...] = pp * g_ref[...]
        g1 = _dotT(tp, pp)
        g3t = _dotT(eh, pp)
        ones_h = jnp.ones((1, H), jnp.float32)
        m_pp = _dotT(ones_h, pp) * (1.0 / H)
        q_pp = _dotT(ones_h, pp * pp)
        tok_all = tok_all_ref[...]
        masks = [tok_all == v for v in range(1, V)]
        mtp = jnp.mean(tp, axis=1, keepdims=True)
        qtp = jnp.sum(tp * tp, axis=1, keepdims=True)
        g2t = g2t_ref[...]
        mu = (_sel6(masks, [mtp[v:v + 1, 0:1] for v in range(1, V)],
                    mtp[0:1, 0:1])
              + meh_ref[...] + m_pp)
        ss = (_sel6(masks, [qtp[v:v + 1, 0:1] for v in range(1, V)],
                    qtp[0:1, 0:1])
              + 2.0 * _sel6(masks, [g1[v:v + 1, :] for v in range(1, V)],
                            g1[0:1, :])
              + 2.0 * _sel6(masks, [g2t[:, v:v + 1] for v in range(1, V)],
                            g2t[:, 0:1])
              + qeh_ref[...] + q_pp + 2.0 * g3t)
        var = ss * (1.0 / H) - mu * mu
        r = lax.rsqrt(var + 1e-5)
        r_ref[...] = r
        mr_ref[...] = mu * r
        rt_ref[...] = r.T

    tok = tok_lane_ref[0]
    iota = lax.broadcasted_iota(jnp.int32, (8, BLK), 0)
    r_row = r_ref[pl.ds(bi, 1), :]
    lhs_ref[0:8, :] = jnp.where(iota == tok, r_row, 0.0)
    lhs_ref[8:9, :] = mr_ref[pl.ds(bi, 1), :]
    lhs_ref[10:11, :] = r_row
    rhs_ref[10:11, :] = ehg_ref[pl.ds(bi, 1), :]
    y = lax.dot_general(lhs_ref[...], rhs_ref[...],
                        (((0,), (0,)), ((), ())),
                        preferred_element_type=jnp.float32)
    ohb = (lax.broadcasted_iota(jnp.int32, (1, B), 1) == bi).astype(jnp.float32)
    r_col = jnp.sum(rt_ref[...] * ohb, axis=1, keepdims=True)
    out_ref[0] = y + ppg_ref[...] * r_col


def kernel(dna_tokens, expr_data, dna_table, pos_enc, expr_W, expr_b,
           dna_proj_W, dna_proj_b, expr_proj_W, expr_proj_b, ln_gamma, ln_beta):
    toks = dna_tokens.astype(jnp.int32)
    tok_lane = toks.reshape(B * NS, 1, BLK)
    tab8 = jnp.pad(dna_table, ((0, 8 - V), (0, 0)))
    xd = jnp.pad(expr_data, ((0, 0), (0, 128 - C)))
    xw = jnp.pad(expr_W, ((0, 0), (0, 128 - C)))
    xb = expr_b.reshape(1, E)
    b2 = (expr_proj_b + dna_proj_b).reshape(1, H)
    g2 = ln_gamma.reshape(1, H)
    bt2 = ln_beta.reshape(1, H)

    grid = (NS, B)
    out = pl.pallas_call(
        _fused_kernel,
        grid=grid,
        in_specs=[
            pl.BlockSpec((1, 1, BLK), lambda si, bi: (bi * NS + si, 0, 0)),
            pl.BlockSpec((B, BLK), lambda si, bi: (0, si)),
            pl.BlockSpec((BLK, D), lambda si, bi: (si, 0)),
            pl.BlockSpec((8, D), lambda si, bi: (0, 0)),
            pl.BlockSpec((H, D), lambda si, bi: (0, 0)),
            pl.BlockSpec((B, 128), lambda si, bi: (0, 0)),
            pl.BlockSpec((E, 128), lambda si, bi: (0, 0)),
            pl.BlockSpec((1, E), lambda si, bi: (0, 0)),
            pl.BlockSpec((H, E), lambda si, bi: (0, 0)),
            pl.BlockSpec((1, H), lambda si, bi: (0, 0)),
            pl.BlockSpec((1, H), lambda si, bi: (0, 0)),
            pl.BlockSpec((1, H), lambda si, bi: (0, 0)),
        ],
        out_specs=pl.BlockSpec((1, BLK, H), lambda si, bi: (bi, si, 0)),
        out_shape=jax.ShapeDtypeStruct((B, S, H), jnp.float32),
        scratch_shapes=[
            pltpu.VMEM((8, H), jnp.float32),
            pltpu.VMEM((B, H), jnp.float32),
            pltpu.VMEM((B, H), jnp.float32),
            pltpu.VMEM((BLK, H), jnp.float32),
            pltpu.VMEM((B, 8), jnp.float32),
            pltpu.VMEM((B, 1), jnp.float32),
            pltpu.VMEM((B, 1), jnp.float32),
            pltpu.VMEM((B, BLK), jnp.float32),
            pltpu.VMEM((B, BLK), jnp.float32),
            pltpu.VMEM((BLK, B), jnp.float32),
            pltpu.VMEM((16, BLK), jnp.float32),
            pltpu.VMEM((16, H), jnp.float32),
        ],
    )(tok_lane, toks, pos_enc, tab8, dna_proj_W, xd, xw, xb, expr_proj_W,
      b2, g2, bt2)
    return out


# speedup vs baseline: 10.1199x; 1.1172x over previous
import jax
import jax.numpy as jnp
from jax import lax
from jax.experimental import pallas as pl
from jax.experimental.pallas import tpu as pltpu

B, S, V, D, C, E, H = 64, 2048, 6, 128, 40, 64, 512
RPS = 2
NSTEP = B // RPS


def _dotT(a, b):
    return lax.dot_general(a, b, (((1,), (1,)), ((), ())),
                           preferred_element_type=jnp.float32)


def _sel6(masks, operands, init):
    acc = init
    for m, o in zip(masks, operands):
        acc = jnp.where(m, o, acc)
    return acc


def _fused_kernel(tok_lane_ref, tok_all_ref, pos_ref, tab_ref, w_ref,
                  xd_ref, xw_ref, xb_ref, pw_ref, b2_ref, g_ref, bt_ref,
                  out_ref,
                  tp_ref, ehg_ref, ppg_ref, mr_ref, r_ref, rt_ref,
                  lhs_a, lhs_b, rhs_a, rhs_b):
    step = pl.program_id(0)

    @pl.when(step == 0)
    def _init():
        g = g_ref[...]
        tp = _dotT(tab_ref[...], w_ref[...])
        tp_ref[...] = tp
        e = _dotT(xd_ref[...], xw_ref[...]) + xb_ref[...]
        eh = _dotT(e, pw_ref[...]) + b2_ref[...]
        ehg_ref[...] = eh * g
        pp = _dotT(pos_ref[...], w_ref[...])
        ppg_ref[...] = pp * g
        g1 = _dotT(tp, pp)
        g2t = _dotT(eh, tp)
        g3t = _dotT(eh, pp)
        ones_h = jnp.ones((1, H), jnp.float32)
        m_pp = _dotT(ones_h, pp) * (1.0 / H)
        q_pp = _dotT(ones_h, pp * pp)
        meh = jnp.mean(eh, axis=1, keepdims=True)
        qeh = jnp.sum(eh * eh, axis=1, keepdims=True)
        tok_all = tok_all_ref[...]
        masks = [tok_all == v for v in range(1, V)]
        mtp = jnp.mean(tp, axis=1, keepdims=True)
        qtp = jnp.sum(tp * tp, axis=1, keepdims=True)
        mu = (_sel6(masks, [mtp[v:v + 1, 0:1] for v in range(1, V)],
                    mtp[0:1, 0:1])
              + meh + m_pp)
        ss = (_sel6(masks, [qtp[v:v + 1, 0:1] for v in range(1, V)],
                    qtp[0:1, 0:1])
              + 2.0 * _sel6(masks, [g1[v:v + 1, :] for v in range(1, V)],
                            g1[0:1, :])
              + 2.0 * _sel6(masks, [g2t[:, v:v + 1] for v in range(1, V)],
                            g2t[:, 0:1])
              + qeh + q_pp + 2.0 * g3t)
        var = ss * (1.0 / H) - mu * mu
        r = lax.rsqrt(var + 1e-5)
        r_ref[...] = r
        mr_ref[...] = mu * r
        rt_ref[...] = r.T
        tpg = tp * g
        for rhs in (rhs_a, rhs_b):
            rhs[0:8, :] = tpg
            rhs[8:9, :] = -g
            rhs[9:10, :] = bt_ref[...]
            rhs[10:16, :] = jnp.zeros((6, H), jnp.float32)
        for lhs in (lhs_a, lhs_b):
            lhs[8:16, :] = jnp.zeros((8, S), jnp.float32)
            lhs[9:10, :] = jnp.ones((1, S), jnp.float32)

    iota = lax.broadcasted_iota(jnp.int32, (8, S), 0)
    iota_b = lax.broadcasted_iota(jnp.int32, (1, B), 1)
    ppg = ppg_ref[...]
    for k, (lhs, rhs) in enumerate(((lhs_a, rhs_a), (lhs_b, rhs_b))):
        bi = step * RPS + k
        tok = tok_lane_ref[k]
        r_row = r_ref[pl.ds(bi, 1), :]
        lhs[0:8, :] = jnp.where(iota == tok, r_row, 0.0)
        lhs[8:9, :] = mr_ref[pl.ds(bi, 1), :]
        lhs[10:11, :] = r_row
        rhs[10:11, :] = ehg_ref[pl.ds(bi, 1), :]
        y = lax.dot_general(lhs[...], rhs[...],
                            (((0,), (0,)), ((), ())),
                            preferred_element_type=jnp.float32)
        ohb = (iota_b == bi).astype(jnp.float32)
        r_col = jnp.sum(rt_ref[...] * ohb, axis=1, keepdims=True)
        out_ref[k] = y + ppg * r_col


def kernel(dna_tokens, expr_data, dna_table, pos_enc, expr_W, expr_b,
           dna_proj_W, dna_proj_b, expr_proj_W, expr_proj_b, ln_gamma, ln_beta):
    toks = dna_tokens.astype(jnp.int32)
    tok_lane = toks.reshape(B, 1, S)
    tab8 = jnp.pad(dna_table, ((0, 8 - V), (0, 0)))
    xd = jnp.pad(expr_data, ((0, 0), (0, 128 - C)))
    xw = jnp.pad(expr_W, ((0, 0), (0, 128 - C)))
    xb = expr_b.reshape(1, E)
    b2 = (expr_proj_b + dna_proj_b).reshape(1, H)
    g2 = ln_gamma.reshape(1, H)
    bt2 = ln_beta.reshape(1, H)

    out = pl.pallas_call(
        _fused_kernel,
        grid=(NSTEP,),
        in_specs=[
            pl.BlockSpec((RPS, 1, S), lambda i: (i, 0, 0)),
            pl.BlockSpec((B, S), lambda i: (0, 0)),
            pl.BlockSpec((S, D), lambda i: (0, 0)),
            pl.BlockSpec((8, D), lambda i: (0, 0)),
            pl.BlockSpec((H, D), lambda i: (0, 0)),
            pl.BlockSpec((B, 128), lambda i: (0, 0)),
            pl.BlockSpec((E, 128), lambda i: (0, 0)),
            pl.BlockSpec((1, E), lambda i: (0, 0)),
            pl.BlockSpec((H, E), lambda i: (0, 0)),
            pl.BlockSpec((1, H), lambda i: (0, 0)),
            pl.BlockSpec((1, H), lambda i: (0, 0)),
            pl.BlockSpec((1, H), lambda i: (0, 0)),
        ],
        out_specs=pl.BlockSpec((RPS, S, H), lambda i: (i, 0, 0)),
        out_shape=jax.ShapeDtypeStruct((B, S, H), jnp.float32),
        scratch_shapes=[
            pltpu.VMEM((8, H), jnp.float32),
            pltpu.VMEM((B, H), jnp.float32),
            pltpu.VMEM((S, H), jnp.float32),
            pltpu.VMEM((B, S), jnp.float32),
            pltpu.VMEM((B, S), jnp.float32),
            pltpu.VMEM((S, B), jnp.float32),
            pltpu.VMEM((16, S), jnp.float32),
            pltpu.VMEM((16, S), jnp.float32),
            pltpu.VMEM((16, H), jnp.float32),
            pltpu.VMEM((16, H), jnp.float32),
        ],
    )(tok_lane, toks, pos_enc, tab8, dna_proj_W, xd, xw, xb, expr_proj_W,
      b2, g2, bt2)
    return out
